# baseline (speedup 1.0000x reference)
import functools

import jax
import jax.numpy as jnp
from jax import lax
from jax.experimental import pallas as pl
from jax.experimental.pallas import tpu as pltpu
from jax.experimental.pallas import tpu_sc as plsc

N_SH = 8
FEAT = 4096
BATCH = 1024
NC, NS, L = 2, 16, 16
NW = NC * NS
JPW = FEAT // NW
KJ = 2
NCHUNK = JPW // KJ
CVR = BATCH // L


def _sc_shuffle(xT, idx_flat, wb, *, interpret=False):
    mesh = plsc.VectorSubcoreMesh(
        core_axis_name="c", subcore_axis_name="s",
        num_cores=NC, num_subcores=NS)

    @functools.partial(
        pl.kernel,
        out_type=jax.ShapeDtypeStruct((FEAT, BATCH), jnp.float32),
        mesh=mesh,
        scratch_types=[
            pltpu.VMEM((JPW * N_SH,), jnp.int32),
            pltpu.VMEM((JPW, L), jnp.float32),
            pltpu.VMEM((KJ * N_SH, BATCH), jnp.float32),
            pltpu.VMEM((KJ, BATCH), jnp.float32),
            pltpu.SemaphoreType.DMA,
        ],
        interpret=interpret,
    )
    def body(xT_hbm, idx_hbm, wb_hbm, out_hbm,
             idx_v, wb_v, rows_v, stage_v, sem):
        wid = lax.axis_index("s") * NC + lax.axis_index("c")
        jbase = wid * JPW
        pltpu.sync_copy(idx_hbm.at[pl.ds(jbase * N_SH, JPW * N_SH)], idx_v)
        pltpu.sync_copy(wb_hbm.at[pl.ds(jbase, JPW)], wb_v)

        @pl.loop(0, NCHUNK)
        def _chunk(c):
            pltpu.async_copy(
                xT_hbm.at[idx_v.at[pl.ds(c * (KJ * N_SH), KJ * N_SH)]],
                rows_v, sem).wait()
            for jj in range(KJ):
                jloc = c * KJ + jj
                wbv = wb_v[jloc]
                ws = [wbv[i] * 0.125 for i in range(N_SH)]
                bsc = wbv[N_SH]
                for ch in range(CVR):
                    acc = jnp.full((L,), bsc, jnp.float32)
                    for i in range(N_SH):
                        acc = acc + rows_v[jj * N_SH + i, pl.ds(ch * L, L)] * ws[i]
                    stage_v[jj, pl.ds(ch * L, L)] = acc
            pltpu.sync_copy(stage_v, out_hbm.at[pl.ds(jbase + c * KJ, KJ)])

    return body(xT, idx_flat, wb)


def kernel(x, weights, bias, perms):
    xT = x.T
    idx_flat = perms.T.reshape(-1)
    wb = jnp.concatenate(
        [weights.T, bias[:, None],
         jnp.zeros((FEAT, L - N_SH - 1), jnp.float32)], axis=1)
    outT = _sc_shuffle(xT, idx_flat, wb)
    return outT.T


# speedup vs baseline: 1.3514x; 1.3514x over previous
import functools

import jax
import jax.numpy as jnp
from jax import lax
from jax.experimental import pallas as pl
from jax.experimental.pallas import tpu as pltpu
from jax.experimental.pallas import tpu_sc as plsc

N_SH = 8
FEAT = 4096
BATCH = 1024
NC, NS, L = 2, 16, 16
NW = NC * NS
JPW = FEAT // NW
KJ = 2
NCHUNK = JPW // KJ
CVR = BATCH // L


def _sc_shuffle(xT, idx_flat, wb, *, interpret=False):
    mesh = plsc.VectorSubcoreMesh(
        core_axis_name="c", subcore_axis_name="s",
        num_cores=NC, num_subcores=NS)

    GR = KJ * N_SH

    @functools.partial(
        pl.kernel,
        out_type=jax.ShapeDtypeStruct((FEAT, BATCH), jnp.float32),
        mesh=mesh,
        scratch_types=[
            pltpu.VMEM((JPW * N_SH,), jnp.int32),
            pltpu.VMEM((JPW, L), jnp.float32),
            pltpu.VMEM((2, GR, BATCH), jnp.float32),
            pltpu.VMEM((2, KJ, BATCH), jnp.float32),
            [pltpu.SemaphoreType.DMA] * 2,
            [pltpu.SemaphoreType.DMA] * 2,
        ],
        interpret=interpret,
    )
    def body(xT_hbm, idx_hbm, wb_hbm, out_hbm,
             idx_v, wb_v, rows_v, stage_v, gsem, ssem):
        wid = lax.axis_index("s") * NC + lax.axis_index("c")
        jbase = wid * JPW
        pltpu.sync_copy(idx_hbm.at[pl.ds(jbase * N_SH, JPW * N_SH)], idx_v)
        pltpu.sync_copy(wb_hbm.at[pl.ds(jbase, JPW)], wb_v)

        def start_gather(c, b):
            pltpu.async_copy(
                xT_hbm.at[idx_v.at[pl.ds(c * GR, GR)]], rows_v.at[b], gsem[b])

        start_gather(0, 0)

        @pl.loop(0, NCHUNK, step=2)
        def _chunk(c):
            for b in range(2):
                cc = c + b
                @pl.when(cc + 1 < NCHUNK)
                def _():
                    start_gather(cc + 1, 1 - b)
                pltpu.make_async_copy(
                    xT_hbm.at[pl.ds(0, GR)], rows_v.at[b], gsem[b]).wait()
                @pl.when(cc >= 2)
                def _():
                    pltpu.make_async_copy(
                        stage_v.at[b], out_hbm.at[pl.ds(jbase, KJ)],
                        ssem[b]).wait()
                for jj in range(KJ):
                    jloc = cc * KJ + jj
                    wbv = wb_v[jloc]
                    ws = [wbv[i] * 0.125 for i in range(N_SH)]
                    bsc = wbv[N_SH]
                    for ch in range(CVR):
                        acc = jnp.full((L,), bsc, jnp.float32)
                        for i in range(N_SH):
                            acc = acc + rows_v[b, jj * N_SH + i,
                                               pl.ds(ch * L, L)] * ws[i]
                        stage_v[b, jj, pl.ds(ch * L, L)] = acc
                pltpu.async_copy(
                    stage_v.at[b], out_hbm.at[pl.ds(jbase + cc * KJ, KJ)],
                    ssem[b])

        for b in range(2):
            pltpu.make_async_copy(
                stage_v.at[b], out_hbm.at[pl.ds(jbase, KJ)], ssem[b]).wait()

    return body(xT, idx_flat, wb)


def kernel(x, weights, bias, perms):
    xT = x.T
    idx_flat = perms.T.reshape(-1)
    wb = jnp.concatenate(
        [weights.T, bias[:, None],
         jnp.zeros((FEAT, L - N_SH - 1), jnp.float32)], axis=1)
    outT = _sc_shuffle(xT, idx_flat, wb)
    return outT.T
